# baseline (speedup 1.0000x reference)
import functools

import jax
import jax.numpy as jnp
from jax import lax
from jax.experimental import pallas as pl
from jax.experimental.pallas import tpu as pltpu
from jax.experimental.pallas import tpu_sc as plsc


def kernel(g, h, r, norm, W):
    B = h.shape[0]
    V, D = W.shape

    info = plsc.get_sparse_core_info()
    NC, NS = info.num_cores, info.num_subcores
    NW = NC * NS
    bpw = B // NW
    K = 16
    NBLK = bpw // K

    idx = h.reshape(B)
    mesh = plsc.VectorSubcoreMesh(core_axis_name="c", subcore_axis_name="s")

    @functools.partial(
        pl.kernel,
        mesh=mesh,
        compiler_params=pltpu.CompilerParams(use_tc_tiling_on_sc=True),
        out_type=jax.ShapeDtypeStruct((B, D), jnp.float32),
        scratch_types=[
            pltpu.VMEM((bpw,), jnp.int32),
            pltpu.VMEM((bpw, D), jnp.float32),
            pltpu.SemaphoreType.DMA,
        ],
    )
    def gather_kernel(idx_hbm, w_hbm, out_hbm, idx_v, rows_v, sem):
        wid = lax.axis_index("s") * NC + lax.axis_index("c")
        base = wid * bpw
        pltpu.sync_copy(idx_hbm.at[pl.ds(base, bpw)], idx_v)

        def block(i, _):
            idx_vec = idx_v[pl.ds(i * K, K)]
            copies = []
            for j in range(K):
                row = idx_vec[j]
                copies.append(
                    pltpu.async_copy(w_hbm.at[row], rows_v.at[i * K + j], sem))
            for c in copies:
                c.wait()
            return 0

        lax.fori_loop(0, NBLK, block, 0)
        pltpu.sync_copy(rows_v, out_hbm.at[pl.ds(base, bpw)])

    return gather_kernel(idx, W)


# speedup vs baseline: 2.4893x; 2.4893x over previous
import functools

import jax
import jax.numpy as jnp
from jax import lax
from jax.experimental import pallas as pl
from jax.experimental.pallas import tpu as pltpu
from jax.experimental.pallas import tpu_sc as plsc


def kernel(g, h, r, norm, W):
    B = h.shape[0]
    V, D = W.shape
    L = 16

    info = plsc.get_sparse_core_info()
    NC, NS = info.num_cores, info.num_subcores
    NW = NC * NS
    bpw = B // NW
    K = 16
    NBLK = bpw // K

    idx = h.reshape(B)
    WT = W.T

    mesh = plsc.VectorSubcoreMesh(core_axis_name="c", subcore_axis_name="s")

    @functools.partial(
        pl.kernel,
        mesh=mesh,
        compiler_params=pltpu.CompilerParams(
            use_tc_tiling_on_sc=True, needs_layout_passes=False),
        out_type=jax.ShapeDtypeStruct((D, B), jnp.float32),
        scratch_types=[
            pltpu.VMEM((bpw,), jnp.int32),
            pltpu.VMEM((K, D, 128), jnp.float32),
            pltpu.VMEM((D, bpw), jnp.float32),
            pltpu.SemaphoreType.DMA,
        ],
    )
    def gather_kernel(idx_hbm, wt_hbm, outt_hbm, idx_v, blk_v, outt_v, sem):
        wid = lax.axis_index("s") * NC + lax.axis_index("c")
        base = wid * bpw
        pltpu.sync_copy(idx_hbm.at[pl.ds(base, bpw)], idx_v)
        lanes = lax.iota(jnp.int32, L)

        def block(i, _):
            idx_vec = idx_v[pl.ds(i * K, K)]
            copies = []
            for j in range(K):
                off = pl.multiple_of(
                    lax.shift_left(lax.shift_right_logical(idx_vec[j], 7), 7),
                    128)
                copies.append(pltpu.async_copy(
                    wt_hbm.at[:, pl.ds(off, 128)], blk_v.at[j], sem))
            for cp in copies:
                cp.wait()
            rm_vec = lax.bitwise_and(idx_vec, 127)
            b_vec = i * K + lanes
            for d in range(D):
                d_vec = jnp.full((L,), d, jnp.int32)
                vals = plsc.load_gather(blk_v, [lanes, d_vec, rm_vec])
                plsc.store_scatter(outt_v, [d_vec, b_vec], vals)
            return 0

        lax.fori_loop(0, NBLK, block, 0)
        pltpu.sync_copy(outt_v, outt_hbm.at[:, pl.ds(base, bpw)])

    return gather_kernel(idx, WT).T


# speedup vs baseline: 2.6364x; 1.0591x over previous
import functools

import jax
import jax.numpy as jnp
from jax import lax
from jax.experimental import pallas as pl
from jax.experimental.pallas import tpu as pltpu
from jax.experimental.pallas import tpu_sc as plsc


def kernel(g, h, r, norm, W):
    B = h.shape[0]
    V, D = W.shape
    L = 16

    info = plsc.get_sparse_core_info()
    NC, NS = info.num_cores, info.num_subcores
    NW = NC * NS
    bpw = B // NW
    K = 16
    NBLK = bpw // K

    idx = h.reshape(B)
    WT = W.T

    mesh = plsc.VectorSubcoreMesh(core_axis_name="c", subcore_axis_name="s")

    def _block_copy(wt_hbm, idx_vec, j, blk_v, sem):
        off = pl.multiple_of(
            lax.shift_left(lax.shift_right_logical(idx_vec[j], 7), 7), 128)
        return pltpu.async_copy(
            wt_hbm.at[:, pl.ds(off, 128)], blk_v.at[j], sem)

    @functools.partial(
        pl.kernel,
        mesh=mesh,
        compiler_params=pltpu.CompilerParams(
            use_tc_tiling_on_sc=True, needs_layout_passes=False),
        out_type=jax.ShapeDtypeStruct((D, B), jnp.float32),
        scratch_types=[
            pltpu.VMEM((bpw,), jnp.int32),
            pltpu.VMEM((K, D, 128), jnp.float32),
            pltpu.VMEM((D, bpw), jnp.float32),
            [pltpu.SemaphoreType.DMA] * K,
        ],
    )
    def gather_kernel(idx_hbm, wt_hbm, outt_hbm, idx_v, blk_v, outt_v, sems):
        wid = lax.axis_index("s") * NC + lax.axis_index("c")
        base = wid * bpw
        pltpu.sync_copy(idx_hbm.at[pl.ds(base, bpw)], idx_v)
        lanes = lax.iota(jnp.int32, L)

        idx_vec0 = idx_v[pl.ds(0, K)]
        for j in range(K):
            _block_copy(wt_hbm, idx_vec0, j, blk_v, sems[j])

        def extract(i, j, rm_vec):
            b_splat = jnp.full((L,), i * K + j, jnp.int32)
            j_splat = jnp.full((L,), j, jnp.int32)
            rm_splat = jnp.broadcast_to(rm_vec[j], (L,))
            for half in range(2):
                d_vec = lanes + half * L
                vals = plsc.load_gather(blk_v, [j_splat, d_vec, rm_splat])
                plsc.store_scatter(outt_v, [d_vec, b_splat], vals)

        def body(i, _):
            idx_vec = idx_v[pl.ds(i * K, K)]
            nxt_vec = idx_v[pl.ds((i + 1) * K, K)]
            rm_vec = lax.bitwise_and(idx_vec, 127)
            for j in range(K):
                pltpu.make_async_copy(
                    wt_hbm.at[:, pl.ds(0, 128)], blk_v.at[j], sems[j]).wait()
                extract(i, j, rm_vec)
                _block_copy(wt_hbm, nxt_vec, j, blk_v, sems[j])
            return 0

        lax.fori_loop(0, NBLK - 1, body, 0)

        i_last = NBLK - 1
        idx_vec = idx_v[pl.ds(i_last * K, K)]
        rm_vec = lax.bitwise_and(idx_vec, 127)
        for j in range(K):
            pltpu.make_async_copy(
                wt_hbm.at[:, pl.ds(0, 128)], blk_v.at[j], sems[j]).wait()
            extract(i_last, j, rm_vec)

        pltpu.sync_copy(outt_v, outt_hbm.at[:, pl.ds(base, bpw)])

    return gather_kernel(idx, WT).T


# speedup vs baseline: 2.6399x; 1.0013x over previous
import functools

import jax
import jax.numpy as jnp
from jax import lax
from jax.experimental import pallas as pl
from jax.experimental.pallas import tpu as pltpu
from jax.experimental.pallas import tpu_sc as plsc


def kernel(g, h, r, norm, W):
    B = h.shape[0]
    V, D = W.shape
    L = 16

    info = plsc.get_sparse_core_info()
    NC, NS = info.num_cores, info.num_subcores
    NW = NC * NS
    bpw = B // NW
    K = 16
    NBLK = bpw // K

    idx = h.reshape(B)
    WT = W.T

    mesh = plsc.VectorSubcoreMesh(core_axis_name="c", subcore_axis_name="s")

    def _block_copy(wt_hbm, idx_vec, j, blk_v, sem):
        off = pl.multiple_of(
            lax.shift_left(lax.shift_right_logical(idx_vec[j], 7), 7), 128)
        for gg in range(D // 8):
            pltpu.async_copy(
                wt_hbm.at[pl.ds(gg * 8, 8), pl.ds(off, 128)],
                blk_v.at[j, pl.ds(gg * 8, 8)], sem)

    @functools.partial(
        pl.kernel,
        mesh=mesh,
        compiler_params=pltpu.CompilerParams(
            use_tc_tiling_on_sc=True, needs_layout_passes=False),
        out_type=jax.ShapeDtypeStruct((D, B), jnp.float32),
        scratch_types=[
            pltpu.VMEM((bpw,), jnp.int32),
            pltpu.VMEM((K, D, 128), jnp.float32),
            pltpu.VMEM((D, bpw), jnp.float32),
            [pltpu.SemaphoreType.DMA] * K,
        ],
    )
    def gather_kernel(idx_hbm, wt_hbm, outt_hbm, idx_v, blk_v, outt_v, sems):
        wid = lax.axis_index("s") * NC + lax.axis_index("c")
        base = wid * bpw
        pltpu.sync_copy(idx_hbm.at[pl.ds(base, bpw)], idx_v)
        lanes = lax.iota(jnp.int32, L)

        idx_vec0 = idx_v[pl.ds(0, K)]
        for j in range(K):
            _block_copy(wt_hbm, idx_vec0, j, blk_v, sems[j])

        def extract(i, j, rm_vec):
            b_splat = jnp.full((L,), i * K + j, jnp.int32)
            j_splat = jnp.full((L,), j, jnp.int32)
            rm_splat = jnp.broadcast_to(rm_vec[j], (L,))
            for half in range(2):
                d_vec = lanes + half * L
                vals = plsc.load_gather(blk_v, [j_splat, d_vec, rm_splat])
                plsc.store_scatter(outt_v, [d_vec, b_splat], vals)

        def body(i, _):
            idx_vec = idx_v[pl.ds(i * K, K)]
            nxt_vec = idx_v[pl.ds((i + 1) * K, K)]
            rm_vec = lax.bitwise_and(idx_vec, 127)
            for j in range(K):
                for gg in range(D // 8):
                    pltpu.make_async_copy(
                        wt_hbm.at[pl.ds(0, 8), pl.ds(0, 128)],
                        blk_v.at[j, pl.ds(gg * 8, 8)], sems[j]).wait()
                extract(i, j, rm_vec)
                _block_copy(wt_hbm, nxt_vec, j, blk_v, sems[j])
            return 0

        lax.fori_loop(0, NBLK - 1, body, 0)

        i_last = NBLK - 1
        idx_vec = idx_v[pl.ds(i_last * K, K)]
        rm_vec = lax.bitwise_and(idx_vec, 127)
        for j in range(K):
            for gg in range(D // 8):
                pltpu.make_async_copy(
                    wt_hbm.at[pl.ds(0, 8), pl.ds(0, 128)],
                    blk_v.at[j, pl.ds(gg * 8, 8)], sems[j]).wait()
            extract(i_last, j, rm_vec)

        pltpu.sync_copy(outt_v, outt_hbm.at[:, pl.ds(base, bpw)])

    return gather_kernel(idx, WT).T
